# baseline (speedup 1.0000x reference)
import functools

import jax
import jax.numpy as jnp
from jax import lax
from jax.experimental import pallas as pl
from jax.experimental.pallas import tpu as pltpu
from jax.experimental.pallas import tpu_sc as plsc

F = 256
HALF = 128
CHUNK = 128
_BN = 1.0 / (1.0 + 1e-5) ** 0.5


def _cdiv(a, b):
    return (a + b - 1) // b


def _mm_body(a_ref, b_ref, o_ref):
    o_ref[...] = jnp.dot(a_ref[...], b_ref[...],
                         preferred_element_type=jnp.float32)


def _mm(a, b):
    m, k = a.shape
    n = b.shape[1]
    bm = min(256, m)
    mp = _cdiv(m, bm) * bm
    if mp != m:
        a = jnp.pad(a, ((0, mp - m), (0, 0)))
    out = pl.pallas_call(
        _mm_body,
        grid=(mp // bm,),
        in_specs=[pl.BlockSpec((bm, k), lambda i: (i, 0)),
                  pl.BlockSpec((k, n), lambda i: (0, 0))],
        out_specs=pl.BlockSpec((bm, n), lambda i: (i, 0)),
        out_shape=jax.ShapeDtypeStruct((mp, n), jnp.float32),
    )(a, b)
    return out[:m] if mp != m else out


@functools.lru_cache(maxsize=None)
def _get_spmm(N, Ep, TN):
    rpt = 8 * _cdiv(N + 1, 128)
    R = 16 * rpt
    ept = Ep // 16
    nch = ept // CHUNK
    full_t = N // rpt
    rem = N - full_t * rpt
    mesh = plsc.VectorSubcoreMesh(core_axis_name="c", subcore_axis_name="s")

    @functools.partial(
        pl.kernel,
        mesh=mesh,
        out_type=(jax.ShapeDtypeStruct((N, HALF), jnp.float32),
                  jax.ShapeDtypeStruct((N, HALF), jnp.float32)),
        scratch_types=[
            pltpu.VMEM((nch, CHUNK), jnp.int32),
            pltpu.VMEM((CHUNK,), jnp.int32),
            pltpu.VMEM((CHUNK,), jnp.int32),
            pltpu.VMEM((CHUNK, HALF), jnp.float32),
            pltpu.VMEM((CHUNK, HALF), jnp.float32),
            pltpu.VMEM((8, HALF), jnp.float32),
            pltpu.VMEM_SHARED((R, HALF), jnp.float32),
            pltpu.SemaphoreType.DMA,
            pltpu.SemaphoreType.DMA,
        ],
    )
    def k(t0, t1, src_hbm, dst2_hbm, out0, out1,
          dst2, si0, si1, r0, r1, zb, acc, sem0, sem1):
        c = lax.axis_index("c")
        s = lax.axis_index("s")

        def zbody(i, carry):
            r = i // 8
            off = (i % 8) * 16
            zb[r, pl.ds(off, 16)] = jnp.zeros((16,), jnp.float32)
            return carry

        lax.fori_loop(0, 64, zbody, 0)

        def zdma(j, carry):
            pltpu.sync_copy(zb, acc.at[pl.ds(s * rpt + j * 8, 8)])
            return carry

        lax.fori_loop(0, rpt // 8, zdma, 0)
        pltpu.sync_copy(dst2_hbm.at[pl.ds(s * nch, nch)], dst2)
        plsc.subcore_barrier()

        def body(tbl, outh):
            def step(j2, carry):
                j = j2 * 2
                pltpu.sync_copy(
                    src_hbm.at[pl.ds(s * ept + j * CHUNK, CHUNK)], si0)
                cp0 = pltpu.async_copy(tbl.at[si0], r0, sem0)
                pltpu.sync_copy(
                    src_hbm.at[pl.ds(s * ept + (j + 1) * CHUNK, CHUNK)], si1)
                cp1 = pltpu.async_copy(tbl.at[si1], r1, sem1)
                cp0.wait()
                pltpu.sync_copy(r0, acc.at[dst2.at[j]], add=True)
                cp1.wait()
                pltpu.sync_copy(r1, acc.at[dst2.at[j + 1]], add=True)
                return carry

            lax.fori_loop(0, nch // 2, step, 0)
            plsc.subcore_barrier()

            @pl.when(s < full_t)
            def _():
                pltpu.sync_copy(acc.at[pl.ds(s * rpt, rpt)],
                                outh.at[pl.ds(s * rpt, rpt)])

            if rem:
                @pl.when(s == full_t)
                def _():
                    pltpu.sync_copy(acc.at[pl.ds(full_t * rpt, rem)],
                                    outh.at[pl.ds(full_t * rpt, rem)])

        @pl.when(c == 0)
        def _():
            body(t0, out0)

        @pl.when(c == 1)
        def _():
            body(t1, out1)

    return k


def _spmm(table, src, dst, N):
    TN = table.shape[0]
    E = src.shape[0]
    Ep = _cdiv(E, 16384) * 16384
    src = jnp.pad(src.astype(jnp.int32), (0, Ep - E))
    dst = jnp.pad(dst.astype(jnp.int32), (0, Ep - E), constant_values=N)
    k = _get_spmm(N, Ep, TN)
    o0, o1 = k(table[:, :HALF], table[:, HALF:], src,
               dst.reshape(Ep // CHUNK, CHUNK))
    return jnp.concatenate([o0, o1], axis=1)


@functools.lru_cache(maxsize=None)
def _get_hist(N, Ep):
    rpt = 8 * _cdiv(N + 1, 128)
    R = 16 * rpt
    eptw = Ep // 32
    nch = eptw // CHUNK
    full_t = N // rpt
    rem = N - full_t * rpt
    mesh = plsc.VectorSubcoreMesh(core_axis_name="c", subcore_axis_name="s")

    @functools.partial(
        pl.kernel,
        mesh=mesh,
        out_type=(jax.ShapeDtypeStruct((N, 16), jnp.float32),
                  jax.ShapeDtypeStruct((N, 16), jnp.float32)),
        scratch_types=[
            pltpu.VMEM((CHUNK,), jnp.int32),
            pltpu.VMEM((CHUNK, 16), jnp.float32),
            pltpu.VMEM((8, 16), jnp.float32),
            pltpu.VMEM_SHARED((R, 16), jnp.float32),
        ],
    )
    def k(dst_hbm, out0, out1, di, ones_v, zb, acc):
        c = lax.axis_index("c")
        s = lax.axis_index("s")

        def fill(i, carry):
            ones_v[i, pl.ds(0, 16)] = jnp.ones((16,), jnp.float32)
            return carry

        lax.fori_loop(0, CHUNK, fill, 0)

        def zbody(i, carry):
            zb[i, pl.ds(0, 16)] = jnp.zeros((16,), jnp.float32)
            return carry

        lax.fori_loop(0, 8, zbody, 0)

        def zdma(j, carry):
            pltpu.sync_copy(zb, acc.at[pl.ds(s * rpt + j * 8, 8)])
            return carry

        lax.fori_loop(0, rpt // 8, zdma, 0)
        plsc.subcore_barrier()

        w = s * 2 + c

        def ch(i, carry):
            off = w * eptw + i * CHUNK
            pltpu.sync_copy(dst_hbm.at[pl.ds(off, CHUNK)], di)
            pltpu.sync_copy(ones_v, acc.at[di], add=True)
            return carry

        lax.fori_loop(0, nch, ch, 0)
        plsc.subcore_barrier()

        def dump(outh):
            @pl.when(s < full_t)
            def _():
                pltpu.sync_copy(acc.at[pl.ds(s * rpt, rpt)],
                                outh.at[pl.ds(s * rpt, rpt)])

            if rem:
                @pl.when(s == full_t)
                def _():
                    pltpu.sync_copy(acc.at[pl.ds(full_t * rpt, rem)],
                                    outh.at[pl.ds(full_t * rpt, rem)])

        @pl.when(c == 0)
        def _():
            dump(out0)

        @pl.when(c == 1)
        def _():
            dump(out1)

    return k


def _hist(dst, N):
    E = dst.shape[0]
    Ep = _cdiv(E, 4096) * 4096
    d = jnp.pad(dst.astype(jnp.int32), (0, Ep - E), constant_values=N)
    o0, o1 = _get_hist(N, Ep)(d)
    return o0[:, 0] + o1[:, 0]


def kernel(x_graph_1, x_graph_2, x_global, params, ei_g1_g1, ei_g2_g2,
           ei_g1_g2, ei_g2_g1, src_glob_g1, dst_glob_g1, src_glob_g2,
           dst_glob_g2, ei_glob_glob, batch_g1, batch_g2, batch_glob,
           slice_placeholder):
    p = params
    N1 = x_graph_1.shape[0]
    N2 = x_graph_2.shape[0]
    NG = x_global.shape[0]

    def bnrelu(h, g, b):
        return jax.nn.relu(g * h * _BN + b)

    h1 = bnrelu(_mm(x_graph_1, p['pre_g1_W']) + p['pre_g1_b'],
                p['pre_bn_g1_g'], p['pre_bn_g1_b'])
    h2 = bnrelu(_mm(x_graph_2, p['pre_g2_W']) + p['pre_g2_b'],
                p['pre_bn_g2_g'], p['pre_bn_g2_b'])
    hg = x_global

    dinv1 = lax.rsqrt(_hist(ei_g1_g1[1], N1) + 1.0)
    dinv2 = lax.rsqrt(_hist(ei_g2_g2[1], N2) + 1.0)
    dinvg = lax.rsqrt(_hist(ei_glob_glob[1], NG) + 1.0)
    c12 = jnp.maximum(_hist(ei_g1_g2[1], N2), 1.0)
    c21 = jnp.maximum(_hist(ei_g2_g1[1], N1), 1.0)
    cb1 = jnp.maximum(_hist(batch_g1, NG), 1.0)
    cb2 = jnp.maximum(_hist(batch_g2, NG), 1.0)

    for l in range(2):
        pre = 'l' + str(l) + '_'
        hw1 = _mm(h1, p[pre + 'gcn11_W']) * dinv1[:, None]
        o11 = (_spmm(hw1, ei_g1_g1[0], ei_g1_g1[1], N1) + hw1) \
            * dinv1[:, None] + p[pre + 'gcn11_b']
        hw2 = _mm(h2, p[pre + 'gcn22_W']) * dinv2[:, None]
        o22 = (_spmm(hw2, ei_g2_g2[0], ei_g2_g2[1], N2) + hw2) \
            * dinv2[:, None] + p[pre + 'gcn22_b']
        hwg = _mm(hg, p[pre + 'gcngg_W']) * dinvg[:, None]
        ogg = (_spmm(hwg, ei_glob_glob[0], ei_glob_glob[1], NG) + hwg) \
            * dinvg[:, None] + p[pre + 'gcngg_b']
        m12 = _spmm(h1, ei_g1_g2[0], ei_g1_g2[1], N2) / c12[:, None]
        o12 = (_mm(m12, p[pre + 'sage12_Wl']) + p[pre + 'sage12_bl']
               + _mm(h2, p[pre + 'sage12_Wr']))
        m21 = _spmm(h2, ei_g2_g1[0], ei_g2_g1[1], N1) / c21[:, None]
        o21 = (_mm(m21, p[pre + 'sage21_Wl']) + p[pre + 'sage21_bl']
               + _mm(h1, p[pre + 'sage21_Wr']))
        og1 = _spmm(_mm(hg, p[pre + 'gatg1_Ws']), src_glob_g1, dst_glob_g1,
                    N1) + p[pre + 'gatg1_b']
        og2 = _spmm(_mm(hg, p[pre + 'gatg2_Ws']), src_glob_g2, dst_glob_g2,
                    N2) + p[pre + 'gatg2_b']
        h1 = _mm(jnp.concatenate([o11, o21, og1], 1),
                 p[pre + 'cat_g1_W']) + p[pre + 'cat_g1_b']
        h2 = _mm(jnp.concatenate([o22, o12, og2], 1),
                 p[pre + 'cat_g2_W']) + p[pre + 'cat_g2_b']
        hg = _mm(ogg, p[pre + 'cat_gl_W']) + p[pre + 'cat_gl_b']

    h1 = bnrelu(_mm(h1, p['post_g1_W']) + p['post_g1_b'],
                p['post_bn_g1_g'], p['post_bn_g1_b'])
    h2 = bnrelu(_mm(h2, p['post_g2_W']) + p['post_g2_b'],
                p['post_bn_g2_g'], p['post_bn_g2_b'])

    ar1 = jnp.arange(N1, dtype=jnp.int32)
    ar2 = jnp.arange(N2, dtype=jnp.int32)
    pool1 = _spmm(h1, ar1, batch_g1, NG) / cb1[:, None]
    pool2 = _spmm(h2, ar2, batch_g2, NG) / cb2[:, None]
    r = jnp.concatenate([pool1, pool2, hg], 1)
    hr = jax.nn.relu(_mm(r, p['lin1_W']) + p['lin1_b'])
    w2 = jnp.pad(p['lin2_W'], ((0, 0), (0, 118)))
    return _mm(hr, w2)[:, :10] + p['lin2_b']


# speedup vs baseline: 1.0116x; 1.0116x over previous
import functools

import jax
import jax.numpy as jnp
from jax import lax
from jax.experimental import pallas as pl
from jax.experimental.pallas import tpu as pltpu
from jax.experimental.pallas import tpu_sc as plsc

F = 256
HALF = 128
CHUNK = 128
_BN = 1.0 / (1.0 + 1e-5) ** 0.5


def _cdiv(a, b):
    return (a + b - 1) // b


def _mm_body(a_ref, b_ref, o_ref):
    o_ref[...] = jnp.dot(a_ref[...], b_ref[...],
                         preferred_element_type=jnp.float32)


def _mm(a, b):
    m, k = a.shape
    n = b.shape[1]
    bm = min(256, m)
    mp = _cdiv(m, bm) * bm
    if mp != m:
        a = jnp.pad(a, ((0, mp - m), (0, 0)))
    out = pl.pallas_call(
        _mm_body,
        grid=(mp // bm,),
        in_specs=[pl.BlockSpec((bm, k), lambda i: (i, 0)),
                  pl.BlockSpec((k, n), lambda i: (0, 0))],
        out_specs=pl.BlockSpec((bm, n), lambda i: (i, 0)),
        out_shape=jax.ShapeDtypeStruct((mp, n), jnp.float32),
    )(a, b)
    return out[:m] if mp != m else out


@functools.lru_cache(maxsize=None)
def _get_spmm(N, Ep, TN):
    rpt = 8 * _cdiv(N + 1, 128)
    R = 16 * rpt
    ept = Ep // 16
    nch = ept // CHUNK
    full_t = N // rpt
    rem = N - full_t * rpt
    mesh = plsc.VectorSubcoreMesh(core_axis_name="c", subcore_axis_name="s")

    @functools.partial(
        pl.kernel,
        mesh=mesh,
        out_type=(jax.ShapeDtypeStruct((N, HALF), jnp.float32),
                  jax.ShapeDtypeStruct((N, HALF), jnp.float32)),
        scratch_types=[
            pltpu.VMEM((CHUNK,), jnp.int32),
            pltpu.VMEM((CHUNK,), jnp.int32),
            pltpu.VMEM((CHUNK,), jnp.int32),
            pltpu.VMEM((CHUNK,), jnp.int32),
            pltpu.VMEM((CHUNK, HALF), jnp.float32),
            pltpu.VMEM((CHUNK, HALF), jnp.float32),
            pltpu.VMEM((8, HALF), jnp.float32),
            pltpu.VMEM_SHARED((R, HALF), jnp.float32),
            pltpu.SemaphoreType.DMA,
            pltpu.SemaphoreType.DMA,
        ],
    )
    def k(t0, t1, src_hbm, dst_hbm, out0, out1,
          si0, si1, di0, di1, r0, r1, zb, acc, sem0, sem1):
        c = lax.axis_index("c")
        s = lax.axis_index("s")

        def zbody(i, carry):
            r = i // 8
            off = (i % 8) * 16
            zb[r, pl.ds(off, 16)] = jnp.zeros((16,), jnp.float32)
            return carry

        lax.fori_loop(0, 64, zbody, 0)

        def zdma(j, carry):
            pltpu.sync_copy(zb, acc.at[pl.ds(s * rpt + j * 8, 8)])
            return carry

        lax.fori_loop(0, rpt // 8, zdma, 0)
        plsc.subcore_barrier()

        def body(tbl, outh):
            def step(j2, carry):
                off = s * ept + j2 * 2 * CHUNK
                pltpu.sync_copy(src_hbm.at[pl.ds(off, CHUNK)], si0)
                cp0 = pltpu.async_copy(tbl.at[si0], r0, sem0)
                pltpu.sync_copy(src_hbm.at[pl.ds(off + CHUNK, CHUNK)], si1)
                cp1 = pltpu.async_copy(tbl.at[si1], r1, sem1)
                pltpu.sync_copy(dst_hbm.at[pl.ds(off, CHUNK)], di0)
                pltpu.sync_copy(dst_hbm.at[pl.ds(off + CHUNK, CHUNK)], di1)
                cp0.wait()
                pltpu.sync_copy(r0, acc.at[di0], add=True)
                cp1.wait()
                pltpu.sync_copy(r1, acc.at[di1], add=True)
                return carry

            lax.fori_loop(0, nch // 2, step, 0)
            plsc.subcore_barrier()

            @pl.when(s < full_t)
            def _():
                pltpu.sync_copy(acc.at[pl.ds(s * rpt, rpt)],
                                outh.at[pl.ds(s * rpt, rpt)])

            if rem:
                @pl.when(s == full_t)
                def _():
                    pltpu.sync_copy(acc.at[pl.ds(full_t * rpt, rem)],
                                    outh.at[pl.ds(full_t * rpt, rem)])

        @pl.when(c == 0)
        def _():
            body(t0, out0)

        @pl.when(c == 1)
        def _():
            body(t1, out1)

    return k


def _spmm(table, src, dst, N):
    TN = table.shape[0]
    E = src.shape[0]
    Ep = _cdiv(E, 16384) * 16384
    src = jnp.pad(src.astype(jnp.int32), (0, Ep - E))
    dst = jnp.pad(dst.astype(jnp.int32), (0, Ep - E), constant_values=N)
    k = _get_spmm(N, Ep, TN)
    o0, o1 = k(table[:, :HALF], table[:, HALF:], src, dst)
    return jnp.concatenate([o0, o1], axis=1)


@functools.lru_cache(maxsize=None)
def _get_hist(N, Ep):
    rpt = 8 * _cdiv(N + 1, 128)
    R = 16 * rpt
    eptw = Ep // 32
    nch = eptw // CHUNK
    full_t = N // rpt
    rem = N - full_t * rpt
    mesh = plsc.VectorSubcoreMesh(core_axis_name="c", subcore_axis_name="s")

    @functools.partial(
        pl.kernel,
        mesh=mesh,
        out_type=(jax.ShapeDtypeStruct((N, 16), jnp.float32),
                  jax.ShapeDtypeStruct((N, 16), jnp.float32)),
        scratch_types=[
            pltpu.VMEM((CHUNK,), jnp.int32),
            pltpu.VMEM((CHUNK, 16), jnp.float32),
            pltpu.VMEM((8, 16), jnp.float32),
            pltpu.VMEM_SHARED((R, 16), jnp.float32),
        ],
    )
    def k(dst_hbm, out0, out1, di, ones_v, zb, acc):
        c = lax.axis_index("c")
        s = lax.axis_index("s")

        def fill(i, carry):
            ones_v[i, pl.ds(0, 16)] = jnp.ones((16,), jnp.float32)
            return carry

        lax.fori_loop(0, CHUNK, fill, 0)

        def zbody(i, carry):
            zb[i, pl.ds(0, 16)] = jnp.zeros((16,), jnp.float32)
            return carry

        lax.fori_loop(0, 8, zbody, 0)

        def zdma(j, carry):
            pltpu.sync_copy(zb, acc.at[pl.ds(s * rpt + j * 8, 8)])
            return carry

        lax.fori_loop(0, rpt // 8, zdma, 0)
        plsc.subcore_barrier()

        w = s * 2 + c

        def ch(i, carry):
            off = w * eptw + i * CHUNK
            pltpu.sync_copy(dst_hbm.at[pl.ds(off, CHUNK)], di)
            pltpu.sync_copy(ones_v, acc.at[di], add=True)
            return carry

        lax.fori_loop(0, nch, ch, 0)
        plsc.subcore_barrier()

        def dump(outh):
            @pl.when(s < full_t)
            def _():
                pltpu.sync_copy(acc.at[pl.ds(s * rpt, rpt)],
                                outh.at[pl.ds(s * rpt, rpt)])

            if rem:
                @pl.when(s == full_t)
                def _():
                    pltpu.sync_copy(acc.at[pl.ds(full_t * rpt, rem)],
                                    outh.at[pl.ds(full_t * rpt, rem)])

        @pl.when(c == 0)
        def _():
            dump(out0)

        @pl.when(c == 1)
        def _():
            dump(out1)

    return k


def _hist(dst, N):
    E = dst.shape[0]
    Ep = _cdiv(E, 4096) * 4096
    d = jnp.pad(dst.astype(jnp.int32), (0, Ep - E), constant_values=N)
    o0, o1 = _get_hist(N, Ep)(d)
    return o0[:, 0] + o1[:, 0]


def kernel(x_graph_1, x_graph_2, x_global, params, ei_g1_g1, ei_g2_g2,
           ei_g1_g2, ei_g2_g1, src_glob_g1, dst_glob_g1, src_glob_g2,
           dst_glob_g2, ei_glob_glob, batch_g1, batch_g2, batch_glob,
           slice_placeholder):
    p = params
    N1 = x_graph_1.shape[0]
    N2 = x_graph_2.shape[0]
    NG = x_global.shape[0]

    def bnrelu(h, g, b):
        return jax.nn.relu(g * h * _BN + b)

    h1 = bnrelu(_mm(x_graph_1, p['pre_g1_W']) + p['pre_g1_b'],
                p['pre_bn_g1_g'], p['pre_bn_g1_b'])
    h2 = bnrelu(_mm(x_graph_2, p['pre_g2_W']) + p['pre_g2_b'],
                p['pre_bn_g2_g'], p['pre_bn_g2_b'])
    hg = x_global

    dinv1 = lax.rsqrt(_hist(ei_g1_g1[1], N1) + 1.0)
    dinv2 = lax.rsqrt(_hist(ei_g2_g2[1], N2) + 1.0)
    dinvg = lax.rsqrt(_hist(ei_glob_glob[1], NG) + 1.0)
    c12 = jnp.maximum(_hist(ei_g1_g2[1], N2), 1.0)
    c21 = jnp.maximum(_hist(ei_g2_g1[1], N1), 1.0)
    cb1 = jnp.maximum(_hist(batch_g1, NG), 1.0)
    cb2 = jnp.maximum(_hist(batch_g2, NG), 1.0)

    for l in range(2):
        pre = 'l' + str(l) + '_'
        hw1 = _mm(h1, p[pre + 'gcn11_W']) * dinv1[:, None]
        o11 = (_spmm(hw1, ei_g1_g1[0], ei_g1_g1[1], N1) + hw1) \
            * dinv1[:, None] + p[pre + 'gcn11_b']
        hw2 = _mm(h2, p[pre + 'gcn22_W']) * dinv2[:, None]
        o22 = (_spmm(hw2, ei_g2_g2[0], ei_g2_g2[1], N2) + hw2) \
            * dinv2[:, None] + p[pre + 'gcn22_b']
        hwg = _mm(hg, p[pre + 'gcngg_W']) * dinvg[:, None]
        ogg = (_spmm(hwg, ei_glob_glob[0], ei_glob_glob[1], NG) + hwg) \
            * dinvg[:, None] + p[pre + 'gcngg_b']
        m12 = _spmm(h1, ei_g1_g2[0], ei_g1_g2[1], N2) / c12[:, None]
        o12 = (_mm(m12, p[pre + 'sage12_Wl']) + p[pre + 'sage12_bl']
               + _mm(h2, p[pre + 'sage12_Wr']))
        m21 = _spmm(h2, ei_g2_g1[0], ei_g2_g1[1], N1) / c21[:, None]
        o21 = (_mm(m21, p[pre + 'sage21_Wl']) + p[pre + 'sage21_bl']
               + _mm(h1, p[pre + 'sage21_Wr']))
        og1 = _spmm(_mm(hg, p[pre + 'gatg1_Ws']), src_glob_g1, dst_glob_g1,
                    N1) + p[pre + 'gatg1_b']
        og2 = _spmm(_mm(hg, p[pre + 'gatg2_Ws']), src_glob_g2, dst_glob_g2,
                    N2) + p[pre + 'gatg2_b']
        h1 = _mm(jnp.concatenate([o11, o21, og1], 1),
                 p[pre + 'cat_g1_W']) + p[pre + 'cat_g1_b']
        h2 = _mm(jnp.concatenate([o22, o12, og2], 1),
                 p[pre + 'cat_g2_W']) + p[pre + 'cat_g2_b']
        hg = _mm(ogg, p[pre + 'cat_gl_W']) + p[pre + 'cat_gl_b']

    h1 = bnrelu(_mm(h1, p['post_g1_W']) + p['post_g1_b'],
                p['post_bn_g1_g'], p['post_bn_g1_b'])
    h2 = bnrelu(_mm(h2, p['post_g2_W']) + p['post_g2_b'],
                p['post_bn_g2_g'], p['post_bn_g2_b'])

    ar1 = jnp.arange(N1, dtype=jnp.int32)
    ar2 = jnp.arange(N2, dtype=jnp.int32)
    pool1 = _spmm(h1, ar1, batch_g1, NG) / cb1[:, None]
    pool2 = _spmm(h2, ar2, batch_g2, NG) / cb2[:, None]
    r = jnp.concatenate([pool1, pool2, hg], 1)
    hr = jax.nn.relu(_mm(r, p['lin1_W']) + p['lin1_b'])
    w2 = jnp.pad(p['lin2_W'], ((0, 0), (0, 118)))
    return _mm(hr, w2)[:, :10] + p['lin2_b']


# speedup vs baseline: 1.7655x; 1.7452x over previous
import functools

import jax
import jax.numpy as jnp
from jax import lax
from jax.experimental import pallas as pl
from jax.experimental.pallas import tpu as pltpu
from jax.experimental.pallas import tpu_sc as plsc

F = 256
HALF = 128
CHUNK = 128
_BN = 1.0 / (1.0 + 1e-5) ** 0.5


def _cdiv(a, b):
    return (a + b - 1) // b


def _mm_body(a_ref, b_ref, o_ref):
    o_ref[...] = jnp.dot(a_ref[...], b_ref[...],
                         preferred_element_type=jnp.float32)


def _mm(a, b):
    m, k = a.shape
    n = b.shape[1]
    bm = min(256, m)
    mp = _cdiv(m, bm) * bm
    if mp != m:
        a = jnp.pad(a, ((0, mp - m), (0, 0)))
    out = pl.pallas_call(
        _mm_body,
        grid=(mp // bm,),
        in_specs=[pl.BlockSpec((bm, k), lambda i: (i, 0)),
                  pl.BlockSpec((k, n), lambda i: (0, 0))],
        out_specs=pl.BlockSpec((bm, n), lambda i: (i, 0)),
        out_shape=jax.ShapeDtypeStruct((mp, n), jnp.float32),
    )(a, b)
    return out[:m] if mp != m else out


@functools.lru_cache(maxsize=None)
def _get_spmm(N, Ep, TN):
    rpt = 8 * _cdiv(N + 1, 128)
    R = 16 * rpt
    ept = Ep // 16
    nch = ept // CHUNK
    full_t = N // rpt
    rem = N - full_t * rpt
    mesh = plsc.VectorSubcoreMesh(core_axis_name="c", subcore_axis_name="s")

    @functools.partial(
        pl.kernel,
        mesh=mesh,
        out_type=(jax.ShapeDtypeStruct((N, HALF), jnp.float32),
                  jax.ShapeDtypeStruct((N, HALF), jnp.float32)),
        scratch_types=[
            pltpu.VMEM((CHUNK,), jnp.int32),
            pltpu.VMEM((CHUNK,), jnp.int32),
            pltpu.VMEM((CHUNK, HALF), jnp.float32),
            pltpu.VMEM((8, HALF), jnp.float32),
            pltpu.VMEM_SHARED((R, HALF), jnp.float32),
            pltpu.SemaphoreType.DMA,
        ],
    )
    def k(t0, t1, src_hbm, dst_hbm, out0, out1, si, di, rows, zb, acc, sem):
        c = lax.axis_index("c")
        s = lax.axis_index("s")

        def zbody(i, carry):
            r = i // 8
            off = (i % 8) * 16
            zb[r, pl.ds(off, 16)] = jnp.zeros((16,), jnp.float32)
            return carry

        lax.fori_loop(0, 64, zbody, 0)

        def zdma(j, carry):
            pltpu.sync_copy(zb, acc.at[pl.ds(s * rpt + j * 8, 8)])
            return carry

        lax.fori_loop(0, rpt // 8, zdma, 0)
        plsc.subcore_barrier()

        def body(tbl, outh):
            def ch(i, carry):
                off = s * ept + i * CHUNK
                pltpu.sync_copy(src_hbm.at[pl.ds(off, CHUNK)], si)
                pltpu.sync_copy(dst_hbm.at[pl.ds(off, CHUNK)], di)
                pltpu.async_copy(tbl.at[si], rows, sem).wait()
                pltpu.sync_copy(rows, acc.at[di], add=True)
                return carry

            lax.fori_loop(0, nch, ch, 0)
            plsc.subcore_barrier()

            @pl.when(s < full_t)
            def _():
                pltpu.sync_copy(acc.at[pl.ds(s * rpt, rpt)],
                                outh.at[pl.ds(s * rpt, rpt)])

            if rem:
                @pl.when(s == full_t)
                def _():
                    pltpu.sync_copy(acc.at[pl.ds(full_t * rpt, rem)],
                                    outh.at[pl.ds(full_t * rpt, rem)])

        @pl.when(c == 0)
        def _():
            body(t0, out0)

        @pl.when(c == 1)
        def _():
            body(t1, out1)

    return k


def _spmm(table, src, dst, N):
    TN = table.shape[0]
    E = src.shape[0]
    Ep = _cdiv(E, 2048) * 2048
    src = jnp.pad(src.astype(jnp.int32), (0, Ep - E))
    dst = jnp.pad(dst.astype(jnp.int32), (0, Ep - E), constant_values=N)
    k = _get_spmm(N, Ep, TN)
    o0, o1 = k(table[:, :HALF], table[:, HALF:], src, dst)
    return jnp.concatenate([o0, o1], axis=1)


@functools.lru_cache(maxsize=None)
def _get_hist(N, Ep):
    rpt = 8 * _cdiv(N + 1, 128)
    R = 16 * rpt
    eptw = Ep // 32
    nch = eptw // CHUNK
    full_t = N // rpt
    rem = N - full_t * rpt
    mesh = plsc.VectorSubcoreMesh(core_axis_name="c", subcore_axis_name="s")

    @functools.partial(
        pl.kernel,
        mesh=mesh,
        out_type=(jax.ShapeDtypeStruct((N, 16), jnp.float32),
                  jax.ShapeDtypeStruct((N, 16), jnp.float32)),
        scratch_types=[
            pltpu.VMEM((CHUNK,), jnp.int32),
            pltpu.VMEM((CHUNK, 16), jnp.float32),
            pltpu.VMEM((8, 16), jnp.float32),
            pltpu.VMEM_SHARED((R, 16), jnp.float32),
        ],
    )
    def k(dst_hbm, out0, out1, di, ones_v, zb, acc):
        c = lax.axis_index("c")
        s = lax.axis_index("s")

        def fill(i, carry):
            ones_v[i, pl.ds(0, 16)] = jnp.ones((16,), jnp.float32)
            return carry

        lax.fori_loop(0, CHUNK, fill, 0)

        def zbody(i, carry):
            zb[i, pl.ds(0, 16)] = jnp.zeros((16,), jnp.float32)
            return carry

        lax.fori_loop(0, 8, zbody, 0)

        def zdma(j, carry):
            pltpu.sync_copy(zb, acc.at[pl.ds(s * rpt + j * 8, 8)])
            return carry

        lax.fori_loop(0, rpt // 8, zdma, 0)
        plsc.subcore_barrier()

        w = s * 2 + c

        def ch(i, carry):
            off = w * eptw + i * CHUNK
            pltpu.sync_copy(dst_hbm.at[pl.ds(off, CHUNK)], di)
            pltpu.sync_copy(ones_v, acc.at[di], add=True)
            return carry

        lax.fori_loop(0, nch, ch, 0)
        plsc.subcore_barrier()

        def dump(outh):
            @pl.when(s < full_t)
            def _():
                pltpu.sync_copy(acc.at[pl.ds(s * rpt, rpt)],
                                outh.at[pl.ds(s * rpt, rpt)])

            if rem:
                @pl.when(s == full_t)
                def _():
                    pltpu.sync_copy(acc.at[pl.ds(full_t * rpt, rem)],
                                    outh.at[pl.ds(full_t * rpt, rem)])

        @pl.when(c == 0)
        def _():
            dump(out0)

        @pl.when(c == 1)
        def _():
            dump(out1)

    return k


def _hist(dst, N):
    E = dst.shape[0]
    Ep = _cdiv(E, 4096) * 4096
    d = jnp.pad(dst.astype(jnp.int32), (0, Ep - E), constant_values=N)
    o0, o1 = _get_hist(N, Ep)(d)
    return o0[:, 0] + o1[:, 0]


def kernel(x_graph_1, x_graph_2, x_global, params, ei_g1_g1, ei_g2_g2,
           ei_g1_g2, ei_g2_g1, src_glob_g1, dst_glob_g1, src_glob_g2,
           dst_glob_g2, ei_glob_glob, batch_g1, batch_g2, batch_glob,
           slice_placeholder):
    p = params
    N1 = x_graph_1.shape[0]
    N2 = x_graph_2.shape[0]
    NG = x_global.shape[0]

    def bnrelu(h, g, b):
        return jax.nn.relu(g * h * _BN + b)

    h1 = bnrelu(_mm(x_graph_1, p['pre_g1_W']) + p['pre_g1_b'],
                p['pre_bn_g1_g'], p['pre_bn_g1_b'])
    h2 = bnrelu(_mm(x_graph_2, p['pre_g2_W']) + p['pre_g2_b'],
                p['pre_bn_g2_g'], p['pre_bn_g2_b'])
    hg = x_global

    dinv1 = lax.rsqrt(_hist(ei_g1_g1[1], N1) + 1.0)
    dinv2 = lax.rsqrt(_hist(ei_g2_g2[1], N2) + 1.0)
    dinvg = lax.rsqrt(_hist(ei_glob_glob[1], NG) + 1.0)
    c12 = jnp.maximum(_hist(ei_g1_g2[1], N2), 1.0)
    c21 = jnp.maximum(_hist(ei_g2_g1[1], N1), 1.0)
    cb1 = jnp.maximum(_hist(batch_g1, NG), 1.0)
    cb2 = jnp.maximum(_hist(batch_g2, NG), 1.0)

    for l in range(2):
        pre = 'l' + str(l) + '_'
        hw1 = _mm(h1, p[pre + 'gcn11_W']) * dinv1[:, None]
        o11 = (_spmm(hw1, ei_g1_g1[0], ei_g1_g1[1], N1) + hw1) \
            * dinv1[:, None] + p[pre + 'gcn11_b']
        hw2 = _mm(h2, p[pre + 'gcn22_W']) * dinv2[:, None]
        o22 = (_spmm(hw2, ei_g2_g2[0], ei_g2_g2[1], N2) + hw2) \
            * dinv2[:, None] + p[pre + 'gcn22_b']
        hwg = _mm(hg, p[pre + 'gcngg_W']) * dinvg[:, None]
        ogg = (_spmm(hwg, ei_glob_glob[0], ei_glob_glob[1], NG) + hwg) \
            * dinvg[:, None] + p[pre + 'gcngg_b']
        m12 = _spmm(h1, ei_g1_g2[0], ei_g1_g2[1], N2) / c12[:, None]
        o12 = (_mm(m12, p[pre + 'sage12_Wl']) + p[pre + 'sage12_bl']
               + _mm(h2, p[pre + 'sage12_Wr']))
        m21 = _spmm(h2, ei_g2_g1[0], ei_g2_g1[1], N1) / c21[:, None]
        o21 = (_mm(m21, p[pre + 'sage21_Wl']) + p[pre + 'sage21_bl']
               + _mm(h1, p[pre + 'sage21_Wr']))
        og1 = _spmm(_mm(hg, p[pre + 'gatg1_Ws']), src_glob_g1, dst_glob_g1,
                    N1) + p[pre + 'gatg1_b']
        og2 = _spmm(_mm(hg, p[pre + 'gatg2_Ws']), src_glob_g2, dst_glob_g2,
                    N2) + p[pre + 'gatg2_b']
        h1 = _mm(jnp.concatenate([o11, o21, og1], 1),
                 p[pre + 'cat_g1_W']) + p[pre + 'cat_g1_b']
        h2 = _mm(jnp.concatenate([o22, o12, og2], 1),
                 p[pre + 'cat_g2_W']) + p[pre + 'cat_g2_b']
        hg = _mm(ogg, p[pre + 'cat_gl_W']) + p[pre + 'cat_gl_b']

    h1 = bnrelu(_mm(h1, p['post_g1_W']) + p['post_g1_b'],
                p['post_bn_g1_g'], p['post_bn_g1_b'])
    h2 = bnrelu(_mm(h2, p['post_g2_W']) + p['post_g2_b'],
                p['post_bn_g2_g'], p['post_bn_g2_b'])

    ar1 = jnp.arange(N1, dtype=jnp.int32)
    ar2 = jnp.arange(N2, dtype=jnp.int32)
    pool1 = _spmm(h1, ar1, batch_g1, NG) / cb1[:, None]
    pool2 = _spmm(h2, ar2, batch_g2, NG) / cb2[:, None]
    r = jnp.concatenate([pool1, pool2, hg], 1)
    hr = jax.nn.relu(_mm(r, p['lin1_W']) + p['lin1_b'])
    w2 = jnp.pad(p['lin2_W'], ((0, 0), (0, 118)))
    return _mm(hr, w2)[:, :10] + p['lin2_b']
